# Initial kernel scaffold; baseline (speedup 1.0000x reference)
import jax
import jax.numpy as jnp
from jax.experimental import pallas as pl


def kernel(features, segment_ids, W_proj, b_proj, ln_w, ln_b):
    raise NotImplementedError("write your pallas kernel here")


# speedup vs baseline: 5.6896x; 5.6896x over previous
import functools

import jax
import jax.numpy as jnp
from jax import lax
from jax.experimental import pallas as pl
from jax.experimental.pallas import tpu as pltpu
from jax.experimental.pallas import tpu_sc as plsc

B = 16
N = 4096
D = 128
S = 16
MAX_OBJECTS = 8
LN_EPS = 1e-5

NC = 2
NS = 16
BPC = B // NC
TPW = (B * N) // (NC * NS)
CH = 128
NCHUNK = TPW // CH


def _sc_segment_sums(features, segment_ids):
    mesh = plsc.VectorSubcoreMesh(core_axis_name="c", subcore_axis_name="s")

    @functools.partial(
        pl.kernel,
        out_type=jax.ShapeDtypeStruct((B, S, D), jnp.float32),
        mesh=mesh,
        scratch_types=[
            pltpu.VMEM((4, CH, D), jnp.float32),
            pltpu.VMEM((TPW,), jnp.int32),
            pltpu.VMEM((NCHUNK, CH), jnp.int32),
            pltpu.VMEM((8, D), jnp.float32),
            pltpu.VMEM_SHARED((BPC * S, D), jnp.float32),
            pltpu.SemaphoreType.DMA,
            pltpu.SemaphoreType.DMA,
        ],
    )
    def sc_kernel(feat_hbm, sid_hbm, out_hbm, featb, sidb, idxb, stage, acc, gsem, ssem):
        c = lax.axis_index("c")
        s = lax.axis_index("s")
        lb = s // 2
        b = c * BPC + lb
        tok0 = (s % 2) * TPW

        zeros16 = jnp.zeros((16,), jnp.float32)
        for i in range(8):
            for j in range(D // 16):
                stage[i, pl.ds(j * 16, 16)] = zeros16
        pltpu.sync_copy(stage, acc.at[pl.ds(s * 8, 8)])

        pltpu.sync_copy(sid_hbm.at[b, pl.ds(tok0, TPW)], sidb)
        for k in range(NCHUNK):
            for j in range(CH // 16):
                idxb[k, pl.ds(j * 16, 16)] = (
                    sidb[pl.ds(k * CH + j * 16, 16)] + lb * S
                )

        plsc.subcore_barrier()

        NBUF = 4
        cps = [
            pltpu.async_copy(
                feat_hbm.at[b, pl.ds(tok0 + k * CH, CH)], featb.at[k], gsem
            )
            for k in range(NBUF)
        ]
        scatters = [None] * NBUF
        for k in range(NCHUNK):
            slot = k % NBUF
            cps[slot].wait()
            scatters[slot] = pltpu.async_copy(
                featb.at[slot], acc.at[idxb.at[k]], ssem, add=True
            )
            nk = k + NBUF
            if nk < NCHUNK:
                scatters[slot].wait()
                scatters[slot] = None
                cps[slot] = pltpu.async_copy(
                    feat_hbm.at[b, pl.ds(tok0 + nk * CH, CH)], featb.at[slot], gsem
                )
        for sc in scatters:
            if sc is not None:
                sc.wait()

        plsc.subcore_barrier()

        pltpu.sync_copy(acc.at[pl.ds(s * 8, 8)], stage)
        pltpu.sync_copy(
            stage, out_hbm.at[c * BPC + s // 2, pl.ds((s % 2) * 8, 8)]
        )

    return sc_kernel(features, segment_ids)


def _tc_finish_body(sums_ref, sid_ref, w_ref, b_ref, lnw_ref, lnb_ref, out_ref):
    sums = sums_ref[...]
    sid = sid_ref[...]

    cols = [
        jnp.sum((sid == s_).astype(jnp.int32), axis=1, keepdims=True)
        for s_ in range(S)
    ]
    counts = jnp.concatenate(cols, axis=1)

    seg_iota = lax.broadcasted_iota(jnp.int32, (B, S), 1)
    key = counts * S + (S - 1 - seg_iota)
    rank = jnp.sum(
        (key[:, :, None] > key[:, None, :]).astype(jnp.int32), axis=1
    )

    means = sums / jnp.maximum(counts, 1)[:, :, None].astype(jnp.float32)

    slot_iota = lax.broadcasted_iota(jnp.int32, (B, MAX_OBJECTS, S), 1)
    sel = jnp.logical_and(
        rank[:, None, :] == slot_iota, (counts > 0)[:, None, :]
    ).astype(jnp.float32)

    pooled = []
    for b_ in range(B):
        pooled.append(
            lax.dot(
                sel[b_], means[b_], preferred_element_type=jnp.float32,
                precision=lax.Precision.HIGHEST,
            )
        )
    pooled2 = jnp.concatenate(pooled, axis=0)

    out = lax.dot_general(
        pooled2, w_ref[...],
        (((1,), (1,)), ((), ())),
        preferred_element_type=jnp.float32,
        precision=lax.Precision.HIGHEST,
    ) + b_ref[...][None, :]

    mu = jnp.mean(out, axis=-1, keepdims=True)
    xc = out - mu
    var = jnp.mean(xc * xc, axis=-1, keepdims=True)
    out = xc * lax.rsqrt(var + LN_EPS) * lnw_ref[...][None, :] + lnb_ref[...][None, :]
    out_ref[...] = out.reshape(B, MAX_OBJECTS, D)


def kernel(features, segment_ids, W_proj, b_proj, ln_w, ln_b):
    segment_ids = segment_ids.astype(jnp.int32)
    sums = _sc_segment_sums(features, segment_ids)
    out = pl.pallas_call(
        _tc_finish_body,
        out_shape=jax.ShapeDtypeStruct((B, MAX_OBJECTS, D), jnp.float32),
    )(sums, segment_ids, W_proj, b_proj, ln_w, ln_b)
    return out
